# Initial kernel scaffold; baseline (speedup 1.0000x reference)
import jax
import jax.numpy as jnp
from jax.experimental import pallas as pl


def kernel(pred_logits, pred_boxes):
    raise NotImplementedError("write your pallas kernel here")


# speedup vs baseline: 390.5409x; 390.5409x over previous
import functools

import jax
import jax.numpy as jnp
from jax import lax
from jax.experimental import pallas as pl
from jax.experimental.pallas import tpu as pltpu
from jax.experimental.pallas import tpu_sc as plsc

_R = 20000
_C = 91
_N = _R * _C
_NC = 2
_NS = 16
_NW = _NC * _NS
_L = 16
_CHUNK = -(-_N // (_NW * _L)) * _L
_NPAD = _CHUNK * _NW
_NEG = -1.0e30


def _sc_scan_body(x_hbm, vals_out, idxs_out, buf, val_s, idx_s):
    cid = lax.axis_index("c")
    sid = lax.axis_index("s")
    wid = sid * _NC + cid
    base = wid * _CHUNK
    pltpu.sync_copy(x_hbm.at[pl.ds(base, _CHUNK)], buf)

    lanes = lax.iota(jnp.int32, 16)

    def step(i, carry):
        bv, bi, cur = carry
        v = buf[pl.ds(i * _L, _L)]
        take = v > bv
        bv = jnp.maximum(bv, v)
        bi = jnp.where(take, cur, bi)
        return bv, bi, cur + _L

    init = (
        jnp.full((_L,), _NEG, jnp.float32),
        jnp.zeros((_L,), jnp.int32),
        base + lanes,
    )
    bv, bi, _ = lax.fori_loop(0, _CHUNK // _L, step, init)

    val_s[...] = bv
    idx_s[...] = bi
    pltpu.sync_copy(val_s, vals_out.at[pl.ds(wid * _L, _L)])
    pltpu.sync_copy(idx_s, idxs_out.at[pl.ds(wid * _L, _L)])


@functools.lru_cache(maxsize=None)
def _build_sc_scan():
    return pl.kernel(
        _sc_scan_body,
        out_type=(
            jax.ShapeDtypeStruct((_NW * _L,), jnp.float32),
            jax.ShapeDtypeStruct((_NW * _L,), jnp.int32),
        ),
        mesh=plsc.VectorSubcoreMesh(
            core_axis_name="c", subcore_axis_name="s",
            num_cores=_NC, num_subcores=_NS,
        ),
        scratch_types=(
            pltpu.VMEM((_CHUNK,), jnp.float32),
            pltpu.VMEM((_L,), jnp.float32),
            pltpu.VMEM((_L,), jnp.int32),
        ),
    )


def _tc_finish_body(v_ref, i_ref, score_ref, label_ref):
    v = v_ref[...]
    idx = i_ref[...]
    m = jnp.max(v)
    sel = jnp.where(v == m, idx, jnp.int32(2**31 - 1))
    mi = jnp.min(sel, keepdims=True).reshape(1, 1)
    score_ref[...] = 1.0 / (1.0 + jnp.exp(-jnp.max(v, keepdims=True).reshape(1, 1)))
    label_ref[...] = mi % _C


def _tc_finish(vals, idxs):
    return pl.pallas_call(
        _tc_finish_body,
        out_shape=(
            jax.ShapeDtypeStruct((1, 1), jnp.float32),
            jax.ShapeDtypeStruct((1, 1), jnp.int32),
        ),
    )(vals, idxs)


def kernel(pred_logits, pred_boxes):
    del pred_boxes
    x = pred_logits[0].reshape(-1)
    xpad = jnp.concatenate(
        [x, jnp.full((_NPAD - _N,), _NEG, jnp.float32)]
    )
    vals, idxs = _build_sc_scan()(xpad)
    score, label = _tc_finish(vals.reshape(4, 128), idxs.reshape(4, 128))
    return (score.reshape(1), label.reshape(1))
